# baseline (speedup 1.0000x reference)
import functools

import jax
import jax.numpy as jnp
from jax import lax
from jax.experimental import pallas as pl
from jax.experimental.pallas import tpu as pltpu
from jax.experimental.pallas import tpu_sc as plsc

N_NODES = 10000
N_EDGES = 320000
NFEAT = 128
NHID = 128

NC, NS = 2, 16
NW = NC * NS
EPW = N_EDGES // NW
E = 128
NCHUNK = -(-EPW // E)
EPW_PAD = NCHUNK * E
N_PAD = 10240
ROWS_PER_TILE = N_PAD // NS
LANES = 16


def _spmm_partials(edata, x):
    mesh = plsc.VectorSubcoreMesh(
        core_axis_name="c", subcore_axis_name="s", num_cores=NC, num_subcores=NS
    )

    @functools.partial(
        pl.kernel,
        out_type=jax.ShapeDtypeStruct((NC, N_PAD, NFEAT), jnp.float32),
        mesh=mesh,
        scratch_types=[
            [pltpu.VMEM((E, NFEAT), jnp.float32) for _ in range(2)],
            [pltpu.VMEM((8, E), jnp.int32) for _ in range(2)],
            pltpu.VMEM_SHARED((N_PAD, NFEAT), jnp.float32),
            [pltpu.SemaphoreType.DMA for _ in range(2)],
            [pltpu.SemaphoreType.DMA for _ in range(2)],
            [pltpu.SemaphoreType.DMA for _ in range(2)],
        ],
    )
    def spmm(edata_hbm, x_hbm, out_hbm, rows, pkt, acc, gsem, ssem, psem):
        c = lax.axis_index("c")
        s = lax.axis_index("s")
        wid = c * NS + s

        def pkt_copy(i, b):
            return pltpu.async_copy(edata_hbm.at[wid, i], pkt[b], psem[b])

        def gather(b):
            pltpu.async_copy(x_hbm.at[pkt[b].at[0]], rows[b], gsem[b])

        def wait_gather(b):
            pltpu.make_async_copy(x_hbm.at[pkt[b].at[0]], rows[b], gsem[b]).wait()

        def scatter(b):
            pltpu.async_copy(rows[b], acc.at[pkt[b].at[1]], ssem[b], add=True)

        def wait_scatter(b):
            pltpu.make_async_copy(rows[b], acc.at[pkt[b].at[1]], ssem[b]).wait()

        def scale(b):
            def grp(g, _):
                wbits = pkt[b][2, pl.ds(g * LANES, LANES)]
                wvec16 = lax.bitcast_convert_type(wbits, jnp.float32)
                for el in range(LANES):
                    wspl = jnp.full((LANES,), wvec16[el], jnp.float32)
                    for j in range(NFEAT // LANES):
                        sl = pl.ds(j * LANES, LANES)
                        e = g * LANES + el
                        rows[b][e, sl] = rows[b][e, sl] * wspl
                return 0

            lax.fori_loop(0, E // LANES, grp, 0)

        zvec = jnp.zeros((LANES,), jnp.float32)

        def zrow(r, _):
            for j in range(NFEAT // LANES):
                rows[0][r, pl.ds(j * LANES, LANES)] = zvec
            return 0

        lax.fori_loop(0, E, zrow, 0)
        for k in range(ROWS_PER_TILE // E):
            pltpu.sync_copy(rows[0], acc.at[pl.ds(s * ROWS_PER_TILE + k * E, E)])
        plsc.subcore_barrier()

        def step(i, cur, first=False, last=False):
            nxt = 1 - cur
            if not first:
                wait_scatter(nxt)
            if not last:
                pd = pkt_copy(i + 1, nxt)
            wait_gather(cur)
            scale(cur)
            if not last:
                pd.wait()
                gather(nxt)
            scatter(cur)

        pkt_copy(0, 0).wait()
        gather(0)

        step(0, 0, first=True)

        def pair(h, _):
            step(2 * h + 1, 1)
            step(2 * h + 2, 0)
            return 0

        lax.fori_loop(0, (NCHUNK - 3) // 2, pair, 0)

        step(NCHUNK - 2, 1)
        step(NCHUNK - 1, 0, last=True)
        wait_scatter(0)

        plsc.subcore_barrier()
        pltpu.sync_copy(
            acc.at[pl.ds(s * ROWS_PER_TILE, ROWS_PER_TILE)],
            out_hbm.at[c, pl.ds(s * ROWS_PER_TILE, ROWS_PER_TILE)],
        )

    return spmm(edata, x)


BM = 1000


def _mm_body(p_ref, w_ref, o_ref):
    agg = p_ref[0] + p_ref[1]
    o_ref[...] = jnp.maximum(
        jnp.dot(agg, w_ref[...], preferred_element_type=jnp.float32), 0.0
    )


def _matmul_relu(partials, W):
    return pl.pallas_call(
        _mm_body,
        grid=(N_NODES // BM,),
        in_specs=[
            pl.BlockSpec((NC, BM, NFEAT), lambda i: (0, i, 0)),
            pl.BlockSpec((NFEAT, NHID), lambda i: (0, 0)),
        ],
        out_specs=pl.BlockSpec((BM, NHID), lambda i: (i, 0)),
        out_shape=jax.ShapeDtypeStruct((N_NODES, NHID), jnp.float32),
    )(partials, W)


def _pack_edges(edge_index, edge_weight):
    pad = EPW_PAD - EPW
    src = jnp.pad(edge_index[1].reshape(NW, EPW), ((0, 0), (0, pad)))
    dst = jnp.pad(edge_index[0].reshape(NW, EPW), ((0, 0), (0, pad)))
    wbits = jnp.pad(
        lax.bitcast_convert_type(edge_weight, jnp.int32).reshape(NW, EPW),
        ((0, 0), (0, pad)),
    )
    zero = jnp.zeros((NW, NCHUNK, E), jnp.int32)
    packed = jnp.stack(
        [
            src.reshape(NW, NCHUNK, E),
            dst.reshape(NW, NCHUNK, E),
            wbits.reshape(NW, NCHUNK, E),
            zero, zero, zero, zero, zero,
        ],
        axis=2,
    )
    return packed


def kernel(edge_index, edge_weight, x, W):
    edata = _pack_edges(edge_index, edge_weight)
    partials = _spmm_partials(edata, x)
    return _matmul_relu(partials, W)


# speedup vs baseline: 1.0954x; 1.0954x over previous
import functools

import jax
import jax.numpy as jnp
from jax import lax
from jax.experimental import pallas as pl
from jax.experimental.pallas import tpu as pltpu
from jax.experimental.pallas import tpu_sc as plsc

N_NODES = 10000
N_EDGES = 320000
NFEAT = 128
NHID = 128

NC, NS = 2, 16
NW = NC * NS
EPW = N_EDGES // NW
E = 128
NCHUNK = -(-EPW // E)
EPW_PAD = NCHUNK * E
N_PAD = 10240
ROWS_PER_TILE = N_PAD // NS
LANES = 16


def _spmm_partials(dst, src, w, x):
    mesh = plsc.VectorSubcoreMesh(
        core_axis_name="c", subcore_axis_name="s", num_cores=NC, num_subcores=NS
    )

    @functools.partial(
        pl.kernel,
        out_type=jax.ShapeDtypeStruct((NC, N_PAD, NFEAT), jnp.float32),
        mesh=mesh,
        scratch_types=[
            [pltpu.VMEM((E, NFEAT), jnp.float32) for _ in range(2)],
            [pltpu.VMEM((E,), jnp.int32) for _ in range(2)],
            [pltpu.VMEM((E,), jnp.int32) for _ in range(2)],
            [pltpu.VMEM((E,), jnp.float32) for _ in range(2)],
            pltpu.VMEM_SHARED((N_PAD, NFEAT), jnp.float32),
            [pltpu.SemaphoreType.DMA for _ in range(2)],
            [pltpu.SemaphoreType.DMA for _ in range(2)],
            [pltpu.SemaphoreType.DMA for _ in range(2)],
        ],
    )
    def spmm(dst_hbm, src_hbm, w_hbm, x_hbm, out_hbm, rows, si, di, wb, acc,
             gsem, ssem, isem):
        c = lax.axis_index("c")
        s = lax.axis_index("s")
        wid = c * NS + s
        ebase = wid * EPW_PAD

        def idx_copies(i, b):
            off = ebase + i * E
            return (
                pltpu.async_copy(src_hbm.at[pl.ds(off, E)], si[b], isem[b]),
                pltpu.async_copy(dst_hbm.at[pl.ds(off, E)], di[b], isem[b]),
                pltpu.async_copy(w_hbm.at[pl.ds(off, E)], wb[b], isem[b]),
            )

        def gather(b):
            pltpu.async_copy(x_hbm.at[si[b]], rows[b], gsem[b])

        def wait_gather(b):
            pltpu.make_async_copy(x_hbm.at[si[b]], rows[b], gsem[b]).wait()

        def scatter(b):
            pltpu.async_copy(rows[b], acc.at[di[b]], ssem[b], add=True)

        def wait_scatter(b):
            pltpu.make_async_copy(rows[b], acc.at[di[b]], ssem[b]).wait()

        def scale(b):
            def grp(g, _):
                wvec16 = wb[b][pl.ds(g * LANES, LANES)]
                for el in range(LANES):
                    wspl = jnp.full((LANES,), wvec16[el], jnp.float32)
                    for j in range(NFEAT // LANES):
                        sl = pl.ds(j * LANES, LANES)
                        e = g * LANES + el
                        rows[b][e, sl] = rows[b][e, sl] * wspl
                return 0

            lax.fori_loop(0, E // LANES, grp, 0)

        zvec = jnp.zeros((LANES,), jnp.float32)

        def zrow(r, _):
            for j in range(NFEAT // LANES):
                rows[0][r, pl.ds(j * LANES, LANES)] = zvec
            return 0

        lax.fori_loop(0, E, zrow, 0)
        for k in range(ROWS_PER_TILE // E):
            pltpu.sync_copy(rows[0], acc.at[pl.ds(s * ROWS_PER_TILE + k * E, E)])
        plsc.subcore_barrier()

        def step(i, cur, first=False, last=False):
            nxt = 1 - cur
            if not first:
                wait_scatter(nxt)
            if not last:
                idx_descs = idx_copies(i + 1, nxt)
            wait_gather(cur)
            scale(cur)
            if not last:
                for d in idx_descs:
                    d.wait()
                gather(nxt)
            scatter(cur)

        for d in idx_copies(0, 0):
            d.wait()
        gather(0)

        step(0, 0, first=True)

        def pair(h, _):
            step(2 * h + 1, 1)
            step(2 * h + 2, 0)
            return 0

        lax.fori_loop(0, (NCHUNK - 3) // 2, pair, 0)

        step(NCHUNK - 2, 1)
        step(NCHUNK - 1, 0, last=True)
        wait_scatter(0)

        plsc.subcore_barrier()
        pltpu.sync_copy(
            acc.at[pl.ds(s * ROWS_PER_TILE, ROWS_PER_TILE)],
            out_hbm.at[c, pl.ds(s * ROWS_PER_TILE, ROWS_PER_TILE)],
        )

    return spmm(dst, src, w, x)


BM = 1000


def _mm_body(p_ref, w_ref, o_ref):
    agg = p_ref[0] + p_ref[1]
    o_ref[...] = jnp.maximum(
        jnp.dot(agg, w_ref[...], preferred_element_type=jnp.float32), 0.0
    )


def _matmul_relu(partials, W):
    return pl.pallas_call(
        _mm_body,
        grid=(N_NODES // BM,),
        in_specs=[
            pl.BlockSpec((NC, BM, NFEAT), lambda i: (0, i, 0)),
            pl.BlockSpec((NFEAT, NHID), lambda i: (0, 0)),
        ],
        out_specs=pl.BlockSpec((BM, NHID), lambda i: (i, 0)),
        out_shape=jax.ShapeDtypeStruct((N_NODES, NHID), jnp.float32),
    )(partials, W)


def _pad_worker(a):
    return jnp.pad(a.reshape(NW, EPW), ((0, 0), (0, EPW_PAD - EPW))).reshape(-1)


def kernel(edge_index, edge_weight, x, W):
    dst = _pad_worker(edge_index[0])
    src = _pad_worker(edge_index[1])
    w = _pad_worker(edge_weight)
    partials = _spmm_partials(dst, src, w, x)
    return _matmul_relu(partials, W)


# speedup vs baseline: 1.4859x; 1.3565x over previous
import functools

import jax
import jax.numpy as jnp
from jax import lax
from jax.experimental import pallas as pl
from jax.experimental.pallas import tpu as pltpu
from jax.experimental.pallas import tpu_sc as plsc

N_NODES = 10000
N_EDGES = 320000
NFEAT = 128
NHID = 128

NC, NS = 2, 16
NW = NC * NS
EPW = N_EDGES // NW
E = 80
NCHUNK = -(-EPW // E)
EPW_PAD = NCHUNK * E
N_PAD = 10240
ROWS_PER_TILE = N_PAD // NS
LANES = 16


def _spmm_partials(dst, sw, x):
    mesh = plsc.VectorSubcoreMesh(
        core_axis_name="c", subcore_axis_name="s", num_cores=NC, num_subcores=NS
    )

    @functools.partial(
        pl.kernel,
        out_type=jax.ShapeDtypeStruct((NC, N_PAD, NFEAT), jnp.float32),
        mesh=mesh,
        scratch_types=[
            [pltpu.VMEM((E, NFEAT), jnp.float32) for _ in range(2)],
            [pltpu.VMEM((2 * E,), jnp.int32) for _ in range(2)],
            [pltpu.VMEM((E,), jnp.int32) for _ in range(2)],
            pltpu.VMEM_SHARED((N_PAD, NFEAT), jnp.float32),
            [pltpu.SemaphoreType.DMA for _ in range(2)],
            [pltpu.SemaphoreType.DMA for _ in range(2)],
            [pltpu.SemaphoreType.DMA for _ in range(2)],
        ],
    )
    def spmm(dst_hbm, sw_hbm, x_hbm, out_hbm, rows, sw, di, acc,
             gsem, ssem, isem):
        c = lax.axis_index("c")
        s = lax.axis_index("s")
        wid = c * NS + s
        dbase = wid * EPW_PAD
        swbase = wid * (2 * EPW_PAD)

        def idx_copies(i, b):
            return (
                pltpu.async_copy(
                    sw_hbm.at[pl.ds(swbase + i * 2 * E, 2 * E)], sw[b], isem[b]
                ),
                pltpu.async_copy(
                    dst_hbm.at[pl.ds(dbase + i * E, E)], di[b], isem[b]
                ),
            )

        def gather(b):
            pltpu.async_copy(x_hbm.at[sw[b].at[pl.ds(0, E)]], rows[b], gsem[b])

        def wait_gather(b):
            pltpu.make_async_copy(
                x_hbm.at[sw[b].at[pl.ds(0, E)]], rows[b], gsem[b]
            ).wait()

        def scatter(b):
            pltpu.async_copy(rows[b], acc.at[di[b]], ssem[b], add=True)

        def wait_scatter(b):
            pltpu.make_async_copy(rows[b], acc.at[di[b]], ssem[b]).wait()

        def scale(b):
            def grp(g, _):
                wbits = sw[b][pl.ds(E + g * LANES, LANES)]
                wvec16 = lax.bitcast_convert_type(wbits, jnp.float32)
                for el in range(LANES):
                    wspl = jnp.full((LANES,), wvec16[el], jnp.float32)
                    for j in range(NFEAT // LANES):
                        sl = pl.ds(j * LANES, LANES)
                        e = g * LANES + el
                        rows[b][e, sl] = rows[b][e, sl] * wspl
                return 0

            lax.fori_loop(0, E // LANES, grp, 0)

        zvec = jnp.zeros((LANES,), jnp.float32)

        def zrow(r, _):
            for j in range(NFEAT // LANES):
                rows[0][r, pl.ds(j * LANES, LANES)] = zvec
            return 0

        lax.fori_loop(0, E, zrow, 0)
        for k in range(ROWS_PER_TILE // E):
            pltpu.sync_copy(rows[0], acc.at[pl.ds(s * ROWS_PER_TILE + k * E, E)])
        plsc.subcore_barrier()

        def step(i, cur, first=False, last=False):
            nxt = 1 - cur
            if not first:
                wait_scatter(nxt)
            if not last:
                idx_descs = idx_copies(i + 1, nxt)
            wait_gather(cur)
            scale(cur)
            if not last:
                for d in idx_descs:
                    d.wait()
                gather(nxt)
            scatter(cur)

        for d in idx_copies(0, 0):
            d.wait()
        gather(0)

        step(0, 0, first=True)

        def pair(h, _):
            step(2 * h + 1, 1)
            step(2 * h + 2, 0)
            return 0

        lax.fori_loop(0, (NCHUNK - 3) // 2, pair, 0)

        step(NCHUNK - 2, 1)
        step(NCHUNK - 1, 0, last=True)
        wait_scatter(0)

        plsc.subcore_barrier()
        pltpu.sync_copy(
            acc.at[pl.ds(s * ROWS_PER_TILE, ROWS_PER_TILE)],
            out_hbm.at[c, pl.ds(s * ROWS_PER_TILE, ROWS_PER_TILE)],
        )

    return spmm(dst, sw, x)


BM = 1000


def _mm_body(p_ref, w_ref, o_ref):
    agg = p_ref[0] + p_ref[1]
    o_ref[...] = jnp.maximum(
        jnp.dot(agg, w_ref[...], preferred_element_type=jnp.float32), 0.0
    )


def _matmul_relu(partials, W):
    return pl.pallas_call(
        _mm_body,
        grid=(N_NODES // BM,),
        in_specs=[
            pl.BlockSpec((NC, BM, NFEAT), lambda i: (0, i, 0)),
            pl.BlockSpec((NFEAT, NHID), lambda i: (0, 0)),
        ],
        out_specs=pl.BlockSpec((BM, NHID), lambda i: (i, 0)),
        out_shape=jax.ShapeDtypeStruct((N_NODES, NHID), jnp.float32),
    )(partials, W)


def kernel(edge_index, edge_weight, x, W):
    dst = edge_index[0]
    src = edge_index[1].reshape(NW, NCHUNK, 1, E)
    wbits = lax.bitcast_convert_type(edge_weight, jnp.int32).reshape(
        NW, NCHUNK, 1, E
    )
    sw = jnp.concatenate([src, wbits], axis=2).reshape(-1)
    partials = _spmm_partials(dst, sw, x)
    return _matmul_relu(partials, W)


# speedup vs baseline: 1.9515x; 1.3133x over previous
import functools

import jax
import jax.numpy as jnp
from jax import lax
from jax.experimental import pallas as pl
from jax.experimental.pallas import tpu as pltpu
from jax.experimental.pallas import tpu_sc as plsc

N_NODES = 10000
N_EDGES = 320000
NFEAT = 128
NHID = 128

NC, NS = 2, 16
NW = NC * NS
EPW = N_EDGES // NW
E = 80
NCHUNK = EPW // E
NR = 3
NI = 4
GROUP = 12
N_PAD = 10240
ROWS_PER_TILE = N_PAD // NS
LANES = 16


def _spmm_partials(dst, src, w, x):
    mesh = plsc.VectorSubcoreMesh(
        core_axis_name="c", subcore_axis_name="s", num_cores=NC, num_subcores=NS
    )

    @functools.partial(
        pl.kernel,
        out_type=jax.ShapeDtypeStruct((NC, N_PAD, NFEAT), jnp.float32),
        mesh=mesh,
        scratch_types=[
            [pltpu.VMEM((E, NFEAT), jnp.float32) for _ in range(NR)],
            [pltpu.VMEM((E,), jnp.int32) for _ in range(NI)],
            [pltpu.VMEM((E,), jnp.int32) for _ in range(NI)],
            [pltpu.VMEM((E,), jnp.float32) for _ in range(NI)],
            pltpu.VMEM_SHARED((N_PAD, NFEAT), jnp.float32),
            [pltpu.SemaphoreType.DMA for _ in range(NR)],
            [pltpu.SemaphoreType.DMA for _ in range(NR)],
            [pltpu.SemaphoreType.DMA for _ in range(NI)],
        ],
    )
    def spmm(dst_hbm, src_hbm, w_hbm, x_hbm, out_hbm, rows, si, di, wb, acc,
             gsem, ssem, isem):
        c = lax.axis_index("c")
        s = lax.axis_index("s")
        wid = c * NS + s
        ebase = wid * EPW

        def idx_copies(i, q):
            off = ebase + i * E
            return (
                pltpu.async_copy(src_hbm.at[pl.ds(off, E)], si[q], isem[q]),
                pltpu.async_copy(dst_hbm.at[pl.ds(off, E)], di[q], isem[q]),
                pltpu.async_copy(w_hbm.at[pl.ds(off, E)], wb[q], isem[q]),
            )

        def wait_idx(q):
            pltpu.make_async_copy(src_hbm.at[pl.ds(0, E)], si[q], isem[q]).wait()
            pltpu.make_async_copy(dst_hbm.at[pl.ds(0, E)], di[q], isem[q]).wait()
            pltpu.make_async_copy(w_hbm.at[pl.ds(0, E)], wb[q], isem[q]).wait()

        def gather(q, r):
            pltpu.async_copy(x_hbm.at[si[q]], rows[r], gsem[r])

        def wait_gather(q, r):
            pltpu.make_async_copy(x_hbm.at[si[q]], rows[r], gsem[r]).wait()

        def scatter(q, r):
            pltpu.async_copy(rows[r], acc.at[di[q]], ssem[r], add=True)

        def wait_scatter(q, r):
            pltpu.make_async_copy(rows[r], acc.at[di[q]], ssem[r]).wait()

        def scale(q, r):
            def grp(g, _):
                wvec16 = wb[q][pl.ds(g * LANES, LANES)]
                for el in range(LANES):
                    wspl = jnp.full((LANES,), wvec16[el], jnp.float32)
                    for j in range(NFEAT // LANES):
                        sl = pl.ds(j * LANES, LANES)
                        e = g * LANES + el
                        rows[r][e, sl] = rows[r][e, sl] * wspl
                return 0

            lax.fori_loop(0, E // LANES, grp, 0)

        zvec = jnp.zeros((LANES,), jnp.float32)

        def zrow(r, _):
            for j in range(NFEAT // LANES):
                rows[0][r, pl.ds(j * LANES, LANES)] = zvec
            return 0

        lax.fori_loop(0, E, zrow, 0)
        for k in range(ROWS_PER_TILE // E):
            pltpu.sync_copy(rows[0], acc.at[pl.ds(s * ROWS_PER_TILE + k * E, E)])
        plsc.subcore_barrier()

        def step(i, im3, im4, drain=True, fill=True, fire=True):
            nx3 = (im3 + 1) % NR
            nx4 = (im4 + 1) % NI
            if drain:
                wait_scatter((im4 + 2) % NI, nx3)
            if fill:
                idx_copies(i + 2, (im4 + 2) % NI)
            wait_gather(im4, im3)
            if fire:
                wait_idx(nx4)
                gather(nx4, nx3)
            scale(im4, im3)
            scatter(im4, im3)

        for d in idx_copies(0, 0):
            d.wait()
        idx_copies(1, 1)
        gather(0, 0)

        step(0, 0, 0, drain=False)
        step(1, 1, 1, drain=False)

        def group(h, _):
            ib = 12 * h + 2
            for k in range(GROUP):
                step(ib + k, (2 + k) % NR, (2 + k) % NI)
            return 0

        lax.fori_loop(0, (NCHUNK - 5) // GROUP, group, 0)

        step(NCHUNK - 3, (NCHUNK - 3) % NR, (NCHUNK - 3) % NI)
        step(NCHUNK - 2, (NCHUNK - 2) % NR, (NCHUNK - 2) % NI, fill=False)
        step(NCHUNK - 1, (NCHUNK - 1) % NR, (NCHUNK - 1) % NI,
             fill=False, fire=False)
        wait_scatter((NCHUNK - 2) % NI, (NCHUNK - 2) % NR)
        wait_scatter((NCHUNK - 1) % NI, (NCHUNK - 1) % NR)

        plsc.subcore_barrier()
        pltpu.sync_copy(
            acc.at[pl.ds(s * ROWS_PER_TILE, ROWS_PER_TILE)],
            out_hbm.at[c, pl.ds(s * ROWS_PER_TILE, ROWS_PER_TILE)],
        )

    return spmm(dst, src, w, x)


BM = 1000


def _mm_body(p_ref, w_ref, o_ref):
    agg = p_ref[0] + p_ref[1]
    o_ref[...] = jnp.maximum(
        jnp.dot(agg, w_ref[...], preferred_element_type=jnp.float32), 0.0
    )


def _matmul_relu(partials, W):
    return pl.pallas_call(
        _mm_body,
        grid=(N_NODES // BM,),
        in_specs=[
            pl.BlockSpec((NC, BM, NFEAT), lambda i: (0, i, 0)),
            pl.BlockSpec((NFEAT, NHID), lambda i: (0, 0)),
        ],
        out_specs=pl.BlockSpec((BM, NHID), lambda i: (i, 0)),
        out_shape=jax.ShapeDtypeStruct((N_NODES, NHID), jnp.float32),
    )(partials, W)


def kernel(edge_index, edge_weight, x, W):
    dst = edge_index[0]
    src = edge_index[1]
    partials = _spmm_partials(dst, src, edge_weight, x)
    return _matmul_relu(partials, W)
